# baseline (speedup 1.0000x reference)
import functools
import math

import jax
import jax.numpy as jnp
from jax import lax
from jax.experimental import pallas as pl
from jax.experimental.pallas import tpu as pltpu
from jax.experimental.pallas import tpu_sc as plsc

CUTOFF = 5.0
KE = 14.3996
N_MOL = 500

NC = 2
NS = 16
L = 16
SUP = 1024
IDX_CHUNK = 128
MOL_PAD = 512

_LOG_CUT = math.log(CUTOFF)
_LN2 = 0.6931471805599453
_SQRT2 = 1.4142135623730951


def _vlog(x):
    bits = lax.bitcast_convert_type(x, jnp.int32)
    e = (bits >> 23) - 127
    m = lax.bitcast_convert_type((bits & 0x007FFFFF) | 0x3F800000, jnp.float32)
    big = m > _SQRT2
    m = jnp.where(big, m * 0.5, m)
    ef = e.astype(jnp.float32) + jnp.where(big, 1.0, 0.0)
    s = (m - 1.0) / (m + 1.0)
    s2 = s * s
    p = 2.0 + s2 * (0.6666666666666666 + s2 * (0.4 + s2 * 0.2857142857142857))
    return ef * _LN2 + s * p


def _make_sc_kernel(n_atoms, a_pad, ept):
    nfull = ept // SUP
    tail = ept - nfull * SUP
    if nfull < 2 or nfull % 2 != 0 or tail % L != 0:
        raise NotImplementedError("pipeline assumes even nfull >= 2")
    a_slice = a_pad // NS
    mesh = plsc.VectorSubcoreMesh(
        core_axis_name="c", subcore_axis_name="s", num_cores=NC, num_subcores=NS
    )

    @functools.partial(
        pl.kernel,
        out_type=jax.ShapeDtypeStruct((NC * MOL_PAD,), jnp.float32),
        mesh=mesh,
        compiler_params=pltpu.CompilerParams(
            needs_layout_passes=False, use_tc_tiling_on_sc=False),
        scratch_types=[
            pltpu.VMEM((SUP,), jnp.int32),
            pltpu.VMEM((SUP,), jnp.int32),
            pltpu.VMEM((SUP, 4), jnp.float32),
            pltpu.VMEM((SUP, 4), jnp.float32),
            pltpu.VMEM((SUP, 3), jnp.float32),
            pltpu.VMEM((SUP,), jnp.int32),
            pltpu.VMEM((SUP,), jnp.int32),
            pltpu.VMEM((SUP, 4), jnp.float32),
            pltpu.VMEM((SUP, 4), jnp.float32),
            pltpu.VMEM((SUP, 3), jnp.float32),
            pltpu.VMEM((a_pad,), jnp.float32),
            pltpu.VMEM((8, IDX_CHUNK), jnp.int32),
            pltpu.VMEM((a_slice,), jnp.float32),
            pltpu.VMEM((a_slice,), jnp.int32),
            pltpu.VMEM((MOL_PAD,), jnp.float32),
            pltpu.VMEM((NS * MOL_PAD,), jnp.float32),
            pltpu.VMEM_SHARED((a_pad,), jnp.float32),
            pltpu.VMEM_SHARED((NS * MOL_PAD,), jnp.float32),
            pltpu.SemaphoreType.DMA,
            pltpu.SemaphoreType.DMA,
            pltpu.SemaphoreType.DMA,
            pltpu.SemaphoreType.DMA,
            pltpu.SemaphoreType.DMA,
            pltpu.SemaphoreType.DMA,
        ],
    )
    def born_sc(atoms_h, ii_h, jj_h, rij_h, im_h, iota_h, out_h,
                *refs):
        (ii_b0, jj_b0, ri_b0, rj_b0, r3_b0,
         ii_b1, jj_b1, ri_b1, rj_b1, r3_b1,
         y_loc, idxa_b, acc_b, im_b, mol_b, molsum_b,
         sh_yacc, sh_mol,
         sem_idx0, sem_idx1, sem_r0, sem_r1, sem_gat0, sem_gat1) = refs
        IIB = (ii_b0, ii_b1)
        JJB = (jj_b0, jj_b1)
        RIB = (ri_b0, ri_b1)
        RJB = (rj_b0, rj_b1)
        R3B = (r3_b0, r3_b1)
        SEMI = (sem_idx0, sem_idx1)
        SEMR = (sem_r0, sem_r1)
        SEMG = (sem_gat0, sem_gat1)
        sem_lin = sem_idx0
        sem_gat = sem_gat0
        cid = lax.axis_index("c")
        sid = lax.axis_index("s")
        w = cid * NS + sid
        zero16 = jnp.zeros((L,), jnp.float32)
        lane = lax.iota(jnp.int32, L)
        c0 = jnp.zeros((L,), jnp.int32)
        c1 = c0 + 1
        c2 = c0 + 2

        def zero_y(i, _):
            y_loc[pl.ds(i * L, L)] = zero16
            return 0

        lax.fori_loop(0, a_pad // L, zero_y, 0)

        def zero_mol(i, _):
            mol_b[pl.ds(i * L, L)] = zero16
            return 0

        lax.fori_loop(0, MOL_PAD // L, zero_mol, 0)

        ebase = w * ept

        def lin_copies(g, b, sz):
            base = ebase + g * SUP
            sl = pl.ds(base, sz)
            d = pl.ds(0, sz)
            return (
                (ii_h.at[sl], IIB[b].at[d], SEMI[b]),
                (jj_h.at[sl], JJB[b].at[d], SEMI[b]),
                (rij_h.at[sl], R3B[b].at[d], SEMR[b]),
            )

        def gat_copies(b, sz):
            out = []
            for t in range(0, sz, IDX_CHUNK):
                c = min(IDX_CHUNK, sz - t)
                s = pl.ds(t, c)
                out.append((atoms_h.at[IIB[b].at[s]], RIB[b].at[s], SEMG[b]))
                out.append((atoms_h.at[JJB[b].at[s]], RJB[b].at[s], SEMG[b]))
            return out

        def issue_lin(g, b, sz=SUP):
            for src, dst, sem in lin_copies(g, b, sz):
                pltpu.async_copy(src, dst, sem)

        def wait_idx(g, b, sz=SUP):
            for src, dst, sem in lin_copies(g, b, sz)[:2]:
                pltpu.make_async_copy(src, dst, sem).wait()

        def wait_r(g, b, sz=SUP):
            for src, dst, sem in lin_copies(g, b, sz)[2:]:
                pltpu.make_async_copy(src, dst, sem).wait()

        def issue_gat(b, sz=SUP):
            for src, dst, sem in gat_copies(b, sz):
                pltpu.async_copy(src, dst, sem)

        def wait_gat(b, sz=SUP):
            for src, dst, sem in gat_copies(b, sz):
                pltpu.make_async_copy(src, dst, sem).wait()

        def compute(b, sz=SUP):
            ii_b, ri_b, rj_b = IIB[b], RIB[b], RJB[b]
            r3_b = R3B[b]

            def step(k, _):
                kbase = k * L
                kk = pl.ds(kbase, L)
                ii = ii_b[kk]
                rows = lane + kbase
                qi = plsc.load_gather(ri_b, [rows, c0])
                ni = plsc.load_gather(ri_b, [rows, c1])
                r0i = plsc.load_gather(ri_b, [rows, c2])
                qj = plsc.load_gather(rj_b, [rows, c0])
                nj = plsc.load_gather(rj_b, [rows, c1])
                r0j = plsc.load_gather(rj_b, [rows, c2])
                x = plsc.load_gather(r3_b, [rows, c0])
                y = plsc.load_gather(r3_b, [rows, c1])
                z = plsc.load_gather(r3_b, [rows, c2])
                d2 = x * x + y * y + z * z
                n = ni + 0.5 * nj
                r0 = r0i + 0.5 * r0j
                q = jnp.abs(qi * qj)
                b_ = q * jnp.exp((n - 1.0) * _vlog(r0)) / n
                h = 0.5 * _vlog(d2)
                pot = b_ * (jnp.exp(-n * h) - jnp.exp(-n * _LOG_CUT))
                pot = jnp.where(d2 <= CUTOFF * CUTOFF, pot, 0.0)
                plsc.addupdate_scatter(y_loc, [ii], pot)
                return 0

            lax.fori_loop(0, sz // L, step, 0)

        issue_lin(0, 0)
        wait_idx(0, 0)
        issue_gat(0)
        issue_lin(1, 1)

        def pair_body(p, _):
            g = 2 * p
            for b in (0, 1):
                gg = g + b
                nb = 1 - b
                wait_idx(gg + 1, nb)
                issue_gat(nb)
                wait_r(gg, b)
                wait_gat(b)
                compute(b)
                issue_lin(gg + 2, b)

            return 0

        lax.fori_loop(0, (nfull - 2) // 2, pair_body, 0)
        wait_idx(nfull - 1, 1)
        issue_gat(1)
        wait_r(nfull - 2, 0)
        wait_gat(0)
        compute(0)
        wait_r(nfull - 1, 1)
        wait_gat(1)
        compute(1)
        if tail:
            issue_lin(nfull, 0, tail)
            wait_idx(nfull, 0, tail)
            issue_gat(0, tail)
            wait_r(nfull, 0, tail)
            wait_gat(0, tail)
            compute(0, tail)

        @pl.when(sid == 0)
        def _():
            pltpu.sync_copy(y_loc, sh_yacc)

        plsc.subcore_barrier()

        @pl.when(sid != 0)
        def _():
            def add_loop(c, _):
                pltpu.async_copy(
                    iota_h.at[pl.ds(c * 8, 8)], idxa_b, sem_lin).wait()
                adds = []
                for r in range(8):
                    adds.append(pltpu.async_copy(
                        y_loc.at[pl.ds((c * 8 + r) * IDX_CHUNK, IDX_CHUNK)],
                        sh_yacc.at[idxa_b.at[r]], sem_gat, add=True))
                for a in adds:
                    a.wait()
                return 0

            lax.fori_loop(0, a_pad // (8 * IDX_CHUNK), add_loop, 0)

        plsc.subcore_barrier()
        abase = sid * a_slice
        pltpu.sync_copy(sh_yacc.at[pl.ds(abase, a_slice)], acc_b)

        pltpu.sync_copy(im_h.at[pl.ds(abase, a_slice)], im_b)

        def mol_loop(k, _):
            kk = pl.ds(k * L, L)
            plsc.addupdate_scatter(mol_b, [im_b[kk]], acc_b[kk])
            return 0

        lax.fori_loop(0, a_slice // L, mol_loop, 0)

        pltpu.sync_copy(mol_b, sh_mol.at[pl.ds(sid * MOL_PAD, MOL_PAD)])
        plsc.subcore_barrier()

        @pl.when(sid == 0)
        def _():
            pltpu.sync_copy(sh_mol, molsum_b)

            def fin_loop(k, _):
                kbase = k * L

                def srow(s, acc):
                    return acc + molsum_b[pl.ds(s * MOL_PAD + kbase, L)]

                acc = lax.fori_loop(1, NS, srow, molsum_b[pl.ds(kbase, L)])
                mol_b[pl.ds(kbase, L)] = acc * (0.5 * KE)
                return 0

            lax.fori_loop(0, MOL_PAD // L, fin_loop, 0)
            pltpu.sync_copy(mol_b, out_h.at[pl.ds(cid * MOL_PAD, MOL_PAD)])

    return born_sc


def kernel(partial_charges, Z, ns, r0s, idx_m, Rij, idx_i, idx_j):
    n_atoms = Z.shape[0]
    n_edges = idx_i.shape[0]
    a_pad = ((n_atoms + NS * L - 1) // (NS * L)) * (NS * L)
    nw = NC * NS
    if n_edges % nw != 0:
        raise NotImplementedError("edge count must divide over 32 tiles")
    ept = n_edges // nw

    q = jnp.squeeze(partial_charges, -1)
    atoms4 = jnp.stack([q, ns, r0s, jnp.zeros_like(q)], axis=1)
    imp = jnp.pad(idx_m, (0, a_pad - n_atoms))
    iota2 = jnp.arange(a_pad, dtype=jnp.int32).reshape(a_pad // IDX_CHUNK,
                                                       IDX_CHUNK)

    out = _make_sc_kernel(n_atoms, a_pad, ept)(
        atoms4, idx_i, idx_j, Rij, imp, iota2)
    return out[:N_MOL] + out[MOL_PAD:MOL_PAD + N_MOL]


# speedup vs baseline: 13.9495x; 13.9495x over previous
import functools
import math

import jax
import jax.numpy as jnp
from jax import lax
from jax.experimental import pallas as pl
from jax.experimental.pallas import tpu as pltpu
from jax.experimental.pallas import tpu_sc as plsc

CUTOFF = 5.0
KE = 14.3996
N_MOL = 500

NC = 2
NS = 16
L = 16
SUP = 1024
IDX_CHUNK = 128
MOL_PAD = 512

_LOG_CUT = math.log(CUTOFF)
_LN2 = 0.6931471805599453
_SQRT2 = 1.4142135623730951


def _vlog(x):
    bits = lax.bitcast_convert_type(x, jnp.int32)
    e = (bits >> 23) - 127
    m = lax.bitcast_convert_type((bits & 0x007FFFFF) | 0x3F800000, jnp.float32)
    big = m > _SQRT2
    m = jnp.where(big, m * 0.5, m)
    ef = e.astype(jnp.float32) + jnp.where(big, 1.0, 0.0)
    s = (m - 1.0) / (m + 1.0)
    s2 = s * s
    p = 2.0 + s2 * (0.6666666666666666 + s2 * (0.4 + s2 * 0.2857142857142857))
    return ef * _LN2 + s * p


def _make_sc_kernel(n_atoms, a_pad, ept):
    nfull = ept // SUP
    tail = ept - nfull * SUP
    if nfull < 2 or nfull % 2 != 0 or tail % L != 0:
        raise NotImplementedError("pipeline assumes even nfull >= 2")
    a_slice = a_pad // NS
    mesh = plsc.VectorSubcoreMesh(
        core_axis_name="c", subcore_axis_name="s", num_cores=NC, num_subcores=NS
    )

    @functools.partial(
        pl.kernel,
        out_type=jax.ShapeDtypeStruct((NC * MOL_PAD,), jnp.float32),
        mesh=mesh,
        compiler_params=pltpu.CompilerParams(
            needs_layout_passes=False, use_tc_tiling_on_sc=False),
        scratch_types=[
            pltpu.VMEM((SUP,), jnp.int32),
            pltpu.VMEM((SUP,), jnp.int32),
            pltpu.VMEM((SUP, 4), jnp.float32),
            pltpu.VMEM((SUP, 4), jnp.float32),
            pltpu.VMEM((SUP,), jnp.float32),
            pltpu.VMEM((SUP,), jnp.float32),
            pltpu.VMEM((SUP,), jnp.float32),
            pltpu.VMEM((SUP,), jnp.int32),
            pltpu.VMEM((SUP,), jnp.int32),
            pltpu.VMEM((SUP, 4), jnp.float32),
            pltpu.VMEM((SUP, 4), jnp.float32),
            pltpu.VMEM((SUP,), jnp.float32),
            pltpu.VMEM((SUP,), jnp.float32),
            pltpu.VMEM((SUP,), jnp.float32),
            pltpu.VMEM((a_pad,), jnp.float32),
            pltpu.VMEM((8, IDX_CHUNK), jnp.int32),
            pltpu.VMEM((a_slice,), jnp.float32),
            pltpu.VMEM((a_slice,), jnp.int32),
            pltpu.VMEM((MOL_PAD,), jnp.float32),
            pltpu.VMEM((NS * MOL_PAD,), jnp.float32),
            pltpu.VMEM_SHARED((a_pad,), jnp.float32),
            pltpu.VMEM_SHARED((NS * MOL_PAD,), jnp.float32),
            pltpu.SemaphoreType.DMA,
            pltpu.SemaphoreType.DMA,
            pltpu.SemaphoreType.DMA,
            pltpu.SemaphoreType.DMA,
            pltpu.SemaphoreType.DMA,
            pltpu.SemaphoreType.DMA,
        ],
    )
    def born_sc(atoms_h, ii_h, jj_h, rx_h, ry_h, rz_h, im_h, iota_h, out_h,
                *refs):
        (ii_b0, jj_b0, ri_b0, rj_b0, rx_b0, ry_b0, rz_b0,
         ii_b1, jj_b1, ri_b1, rj_b1, rx_b1, ry_b1, rz_b1,
         y_loc, idxa_b, acc_b, im_b, mol_b, molsum_b,
         sh_yacc, sh_mol,
         sem_idx0, sem_idx1, sem_r0, sem_r1, sem_gat0, sem_gat1) = refs
        IIB = (ii_b0, ii_b1)
        JJB = (jj_b0, jj_b1)
        RIB = (ri_b0, ri_b1)
        RJB = (rj_b0, rj_b1)
        RXB = (rx_b0, rx_b1)
        RYB = (ry_b0, ry_b1)
        RZB = (rz_b0, rz_b1)
        SEMI = (sem_idx0, sem_idx1)
        SEMR = (sem_r0, sem_r1)
        SEMG = (sem_gat0, sem_gat1)
        sem_lin = sem_idx0
        sem_gat = sem_gat0
        cid = lax.axis_index("c")
        sid = lax.axis_index("s")
        w = cid * NS + sid
        zero16 = jnp.zeros((L,), jnp.float32)
        lane = lax.iota(jnp.int32, L)
        c0 = jnp.zeros((L,), jnp.int32)
        c1 = c0 + 1
        c2 = c0 + 2

        def zero_y(i, _):
            y_loc[pl.ds(i * L, L)] = zero16
            return 0

        lax.fori_loop(0, a_pad // L, zero_y, 0)

        def zero_mol(i, _):
            mol_b[pl.ds(i * L, L)] = zero16
            return 0

        lax.fori_loop(0, MOL_PAD // L, zero_mol, 0)

        ebase = w * ept

        def lin_copies(g, b, sz):
            base = ebase + g * SUP
            sl = pl.ds(base, sz)
            d = pl.ds(0, sz)
            return (
                (ii_h.at[sl], IIB[b].at[d], SEMI[b]),
                (jj_h.at[sl], JJB[b].at[d], SEMI[b]),
                (rx_h.at[sl], RXB[b].at[d], SEMR[b]),
                (ry_h.at[sl], RYB[b].at[d], SEMR[b]),
                (rz_h.at[sl], RZB[b].at[d], SEMR[b]),
            )

        def gat_copies(b, sz):
            out = []
            for t in range(0, sz, IDX_CHUNK):
                c = min(IDX_CHUNK, sz - t)
                s = pl.ds(t, c)
                out.append((atoms_h.at[IIB[b].at[s]], RIB[b].at[s], SEMG[b]))
                out.append((atoms_h.at[JJB[b].at[s]], RJB[b].at[s], SEMG[b]))
            return out

        def issue_lin(g, b, sz=SUP):
            for src, dst, sem in lin_copies(g, b, sz):
                pltpu.async_copy(src, dst, sem)

        def wait_idx(g, b, sz=SUP):
            for src, dst, sem in lin_copies(g, b, sz)[:2]:
                pltpu.make_async_copy(src, dst, sem).wait()

        def wait_r(g, b, sz=SUP):
            for src, dst, sem in lin_copies(g, b, sz)[2:]:
                pltpu.make_async_copy(src, dst, sem).wait()

        def issue_gat(b, sz=SUP):
            for src, dst, sem in gat_copies(b, sz):
                pltpu.async_copy(src, dst, sem)

        def wait_gat(b, sz=SUP):
            for src, dst, sem in gat_copies(b, sz):
                pltpu.make_async_copy(src, dst, sem).wait()

        def compute(b, sz=SUP):
            ii_b, ri_b, rj_b = IIB[b], RIB[b], RJB[b]
            rx_b, ry_b, rz_b = RXB[b], RYB[b], RZB[b]

            def step(k, _):
                kbase = k * L
                kk = pl.ds(kbase, L)
                ii = ii_b[kk]
                rows = lane + kbase
                qi = plsc.load_gather(ri_b, [rows, c0])
                ni = plsc.load_gather(ri_b, [rows, c1])
                r0i = plsc.load_gather(ri_b, [rows, c2])
                qj = plsc.load_gather(rj_b, [rows, c0])
                nj = plsc.load_gather(rj_b, [rows, c1])
                r0j = plsc.load_gather(rj_b, [rows, c2])
                x = rx_b[kk]
                y = ry_b[kk]
                z = rz_b[kk]
                d2 = x * x + y * y + z * z
                n = ni + 0.5 * nj
                r0 = r0i + 0.5 * r0j
                q = jnp.abs(qi * qj)
                b_ = q * jnp.exp((n - 1.0) * _vlog(r0)) / n
                h = 0.5 * _vlog(d2)
                pot = b_ * (jnp.exp(-n * h) - jnp.exp(-n * _LOG_CUT))
                pot = jnp.where(d2 <= CUTOFF * CUTOFF, pot, 0.0)
                plsc.addupdate_scatter(y_loc, [ii], pot)
                return 0

            lax.fori_loop(0, sz // L, step, 0)

        issue_lin(0, 0)
        wait_idx(0, 0)
        issue_gat(0)
        issue_lin(1, 1)

        def pair_body(p, _):
            g = 2 * p
            for b in (0, 1):
                gg = g + b
                nb = 1 - b
                wait_idx(gg + 1, nb)
                issue_gat(nb)
                wait_r(gg, b)
                wait_gat(b)
                compute(b)
                issue_lin(gg + 2, b)

            return 0

        lax.fori_loop(0, (nfull - 2) // 2, pair_body, 0)
        wait_idx(nfull - 1, 1)
        issue_gat(1)
        wait_r(nfull - 2, 0)
        wait_gat(0)
        compute(0)
        wait_r(nfull - 1, 1)
        wait_gat(1)
        compute(1)
        if tail:
            issue_lin(nfull, 0, tail)
            wait_idx(nfull, 0, tail)
            issue_gat(0, tail)
            wait_r(nfull, 0, tail)
            wait_gat(0, tail)
            compute(0, tail)

        @pl.when(sid == 0)
        def _():
            pltpu.sync_copy(y_loc, sh_yacc)

        plsc.subcore_barrier()

        @pl.when(sid != 0)
        def _():
            def add_loop(c, _):
                pltpu.async_copy(
                    iota_h.at[pl.ds(c * 8, 8)], idxa_b, sem_lin).wait()
                adds = []
                for r in range(8):
                    adds.append(pltpu.async_copy(
                        y_loc.at[pl.ds((c * 8 + r) * IDX_CHUNK, IDX_CHUNK)],
                        sh_yacc.at[idxa_b.at[r]], sem_gat, add=True))
                for a in adds:
                    a.wait()
                return 0

            lax.fori_loop(0, a_pad // (8 * IDX_CHUNK), add_loop, 0)

        plsc.subcore_barrier()
        abase = sid * a_slice
        pltpu.sync_copy(sh_yacc.at[pl.ds(abase, a_slice)], acc_b)

        pltpu.sync_copy(im_h.at[pl.ds(abase, a_slice)], im_b)

        def mol_loop(k, _):
            kk = pl.ds(k * L, L)
            plsc.addupdate_scatter(mol_b, [im_b[kk]], acc_b[kk])
            return 0

        lax.fori_loop(0, a_slice // L, mol_loop, 0)

        pltpu.sync_copy(mol_b, sh_mol.at[pl.ds(sid * MOL_PAD, MOL_PAD)])
        plsc.subcore_barrier()

        @pl.when(sid == 0)
        def _():
            pltpu.sync_copy(sh_mol, molsum_b)

            def fin_loop(k, _):
                kbase = k * L

                def srow(s, acc):
                    return acc + molsum_b[pl.ds(s * MOL_PAD + kbase, L)]

                acc = lax.fori_loop(1, NS, srow, molsum_b[pl.ds(kbase, L)])
                mol_b[pl.ds(kbase, L)] = acc * (0.5 * KE)
                return 0

            lax.fori_loop(0, MOL_PAD // L, fin_loop, 0)
            pltpu.sync_copy(mol_b, out_h.at[pl.ds(cid * MOL_PAD, MOL_PAD)])

    return born_sc


def kernel(partial_charges, Z, ns, r0s, idx_m, Rij, idx_i, idx_j):
    n_atoms = Z.shape[0]
    n_edges = idx_i.shape[0]
    a_pad = ((n_atoms + NS * L - 1) // (NS * L)) * (NS * L)
    nw = NC * NS
    if n_edges % nw != 0:
        raise NotImplementedError("edge count must divide over 32 tiles")
    ept = n_edges // nw

    q = jnp.squeeze(partial_charges, -1)
    atoms4 = jnp.stack([q, ns, r0s, jnp.zeros_like(q)], axis=1)
    imp = jnp.pad(idx_m, (0, a_pad - n_atoms))
    iota2 = jnp.arange(a_pad, dtype=jnp.int32).reshape(a_pad // IDX_CHUNK,
                                                       IDX_CHUNK)

    out = _make_sc_kernel(n_atoms, a_pad, ept)(
        atoms4, idx_i, idx_j, Rij[:, 0], Rij[:, 1], Rij[:, 2], imp, iota2)
    return out[:N_MOL] + out[MOL_PAD:MOL_PAD + N_MOL]
